# Initial kernel scaffold; baseline (speedup 1.0000x reference)
import jax
import jax.numpy as jnp
from jax.experimental import pallas as pl


def kernel(features, labels, centers):
    raise NotImplementedError("write your pallas kernel here")


# speedup vs baseline: 1.1334x; 1.1334x over previous
import functools

import jax
import jax.numpy as jnp
from jax import lax
from jax.experimental import pallas as pl
from jax.experimental.pallas import tpu as pltpu
from jax.experimental.pallas import tpu_sc as plsc

_NUM_CLASSES = 100000
_D = 128
_B = 4096
_LAMBDA = 0.003

_NC = 2
_NS = 16
_L = 16
_NW = _NC * _NS
_BPW = _B // _NW
_COLS = _D // _L

_mesh = plsc.VectorSubcoreMesh(core_axis_name="c", subcore_axis_name="s")


@functools.partial(
    pl.kernel,
    out_type=jax.ShapeDtypeStruct((_NW, _L), jnp.float32),
    mesh=_mesh,
    scratch_types=[
        pltpu.VMEM((_BPW,), jnp.int32),
        pltpu.VMEM((_BPW, _D), jnp.float32),
        pltpu.VMEM((_BPW, _D), jnp.float32),
        pltpu.VMEM((_L,), jnp.float32),
        pltpu.SemaphoreType.DMA,
        pltpu.SemaphoreType.DMA,
    ],
)
def _sc_partial_sums(feat_hbm, lab_hbm, cent_hbm, out_hbm,
                     idx_v, rows_v, feat_v, acc_v, sem_g, sem_f):
    wid = lax.axis_index("s") * _NC + lax.axis_index("c")
    base = wid * _BPW

    pltpu.sync_copy(lab_hbm.at[pl.ds(base, _BPW)], idx_v)
    cp_f = pltpu.async_copy(feat_hbm.at[pl.ds(base, _BPW)], feat_v, sem_f)
    cp_g = pltpu.async_copy(cent_hbm.at[idx_v], rows_v, sem_g)
    cp_f.wait()
    cp_g.wait()

    def row_body(i, acc):
        for j in range(_COLS):
            f = feat_v[i, pl.ds(j * _L, _L)]
            c = rows_v[i, pl.ds(j * _L, _L)]
            d = f - c
            acc = acc + d * d
        return acc

    acc = lax.fori_loop(0, _BPW, row_body, jnp.zeros((_L,), jnp.float32))
    acc_v[...] = acc
    pltpu.sync_copy(acc_v, out_hbm.at[wid])


def _tc_finish(p_ref, o_ref):
    o_ref[0, 0] = jnp.sum(p_ref[...]) * (_LAMBDA / _B)


_finish_call = pl.pallas_call(
    _tc_finish,
    out_shape=jax.ShapeDtypeStruct((1, 1), jnp.float32),
    out_specs=pl.BlockSpec(memory_space=pltpu.SMEM),
)


@jax.jit
def kernel(features, labels, centers):
    partials = _sc_partial_sums(features, labels.astype(jnp.int32), centers)
    return _finish_call(partials)[0, 0]
